# Initial kernel scaffold; baseline (speedup 1.0000x reference)
import jax
import jax.numpy as jnp
from jax.experimental import pallas as pl


def kernel(x, edge_index, W1, b1, W2, b2):
    raise NotImplementedError("write your pallas kernel here")


# speedup vs baseline: 32.7649x; 32.7649x over previous
import functools

import jax
import jax.numpy as jnp
from jax import lax
from jax.experimental import pallas as pl
from jax.experimental.pallas import tpu as pltpu
import jax.experimental.pallas.tpu_sc as plsc

N = 10000
E = 320000
IN_DIM = 128
HID = 16
NUM_CLASSES = 7

NP = 10240
NTILES = 32
EDGES_PER_TILE = NP
E_PAD = NTILES * EDGES_PER_TILE
IDX_ROWS = E_PAD // 128
ROWS_PER_TILE = IDX_ROWS // NTILES
CHUNK_ROWS = 8
NCHUNKS = ROWS_PER_TILE // CHUNK_ROWS
NODES_PER_TILE = NP // 16

_MESH = plsc.VectorSubcoreMesh(core_axis_name="c", subcore_axis_name="s")


@functools.partial(
    pl.kernel,
    out_type=jax.ShapeDtypeStruct((2 * NP,), jnp.float32),
    mesh=_MESH,
    scratch_types=[
        pltpu.VMEM((ROWS_PER_TILE, 128), jnp.int32),
        pltpu.VMEM((128,), jnp.float32),
        pltpu.VMEM_SHARED((NP,), jnp.float32),
    ],
)
def _sc_degree(dst_hbm, zeros_hbm, ones_hbm, out_hbm, didx, ones_v, cnt):
    cid = lax.axis_index("c")
    sid = lax.axis_index("s")
    wid = cid * 16 + sid
    pltpu.sync_copy(zeros_hbm.at[pl.ds(sid * NODES_PER_TILE, NODES_PER_TILE)],
                    cnt.at[pl.ds(sid * NODES_PER_TILE, NODES_PER_TILE)])
    pltpu.sync_copy(ones_hbm, ones_v)
    pltpu.sync_copy(dst_hbm.at[pl.ds(wid * ROWS_PER_TILE, ROWS_PER_TILE)], didx)
    plsc.subcore_barrier()

    def body(j, carry):
        pltpu.sync_copy(ones_v, cnt.at[didx.at[j]], add=True)
        return carry

    lax.fori_loop(0, ROWS_PER_TILE, body, 0)
    plsc.subcore_barrier()
    pltpu.sync_copy(cnt.at[pl.ds(sid * NODES_PER_TILE, NODES_PER_TILE)],
                    out_hbm.at[pl.ds(cid * NP + sid * NODES_PER_TILE,
                                     NODES_PER_TILE)])


@functools.partial(
    pl.kernel,
    out_type=jax.ShapeDtypeStruct((2 * NP, HID), jnp.float32),
    mesh=_MESH,
    scratch_types=[
        pltpu.VMEM((ROWS_PER_TILE, 128), jnp.int32),
        pltpu.VMEM((ROWS_PER_TILE, 128), jnp.int32),
        pltpu.VMEM((CHUNK_ROWS * 128, HID), jnp.float32),
        pltpu.VMEM_SHARED((NP, HID), jnp.float32),
        pltpu.SemaphoreType.DMA,
    ],
    compiler_params=pltpu.CompilerParams(use_tc_tiling_on_sc=False),
)
def _sc_agg(table_hbm, src_hbm, dst_hbm, zeros_hbm, out_hbm,
            sidx, didx, rows, acc, sem):
    cid = lax.axis_index("c")
    sid = lax.axis_index("s")
    wid = cid * 16 + sid
    pltpu.sync_copy(zeros_hbm.at[pl.ds(sid * NODES_PER_TILE, NODES_PER_TILE)],
                    acc.at[pl.ds(sid * NODES_PER_TILE, NODES_PER_TILE)])
    pltpu.sync_copy(src_hbm.at[pl.ds(wid * ROWS_PER_TILE, ROWS_PER_TILE)], sidx)
    pltpu.sync_copy(dst_hbm.at[pl.ds(wid * ROWS_PER_TILE, ROWS_PER_TILE)], didx)
    plsc.subcore_barrier()

    def chunk(c, carry):
        base = c * CHUNK_ROWS
        cps = []
        for j in range(CHUNK_ROWS):
            cps.append(pltpu.async_copy(
                table_hbm.at[sidx.at[base + j]],
                rows.at[pl.ds(j * 128, 128)], sem))
        for cp in cps:
            cp.wait()
        for j in range(CHUNK_ROWS):
            pltpu.sync_copy(rows.at[pl.ds(j * 128, 128)],
                            acc.at[didx.at[base + j]], add=True)
        return carry

    lax.fori_loop(0, NCHUNKS, chunk, 0)
    plsc.subcore_barrier()
    pltpu.sync_copy(acc.at[pl.ds(sid * NODES_PER_TILE, NODES_PER_TILE)],
                    out_hbm.at[pl.ds(cid * NP + sid * NODES_PER_TILE,
                                     NODES_PER_TILE)])


_R = 512
_GRID = NP // _R


def _tc1_body(cnt_ref, x_ref, w1_ref, hs_ref):
    c = cnt_ref[...]
    dinv = lax.rsqrt(1.0 + c[:, 0:1] + c[:, 1:2])
    h = jnp.dot(x_ref[...], w1_ref[...], preferred_element_type=jnp.float32)
    hs_ref[...] = dinv * h


def _tc1(cnt_t, x_p, W1):
    return pl.pallas_call(
        _tc1_body,
        grid=(_GRID,),
        in_specs=[
            pl.BlockSpec((_R, 2), lambda r: (r, 0)),
            pl.BlockSpec((_R, IN_DIM), lambda r: (r, 0)),
            pl.BlockSpec((IN_DIM, HID), lambda r: (0, 0)),
        ],
        out_specs=pl.BlockSpec((_R, HID), lambda r: (r, 0)),
        out_shape=jax.ShapeDtypeStruct((NP, HID), jnp.float32),
    )(cnt_t, x_p, W1)


def _tc2_body(a_ref, hs_ref, cnt_ref, b1_ref, hs1_ref):
    c = cnt_ref[...]
    dinv = lax.rsqrt(1.0 + c[:, 0:1] + c[:, 1:2])
    a = a_ref[...]
    h1 = jnp.maximum(dinv * (a[0] + a[1] + hs_ref[...]) + b1_ref[...], 0.0)
    hs1_ref[...] = dinv * h1


def _tc2(acc, hs, cnt_t, b1):
    return pl.pallas_call(
        _tc2_body,
        grid=(_GRID,),
        in_specs=[
            pl.BlockSpec((2, _R, HID), lambda r: (0, r, 0)),
            pl.BlockSpec((_R, HID), lambda r: (r, 0)),
            pl.BlockSpec((_R, 2), lambda r: (r, 0)),
            pl.BlockSpec((HID,), lambda r: (0,)),
        ],
        out_specs=pl.BlockSpec((_R, HID), lambda r: (r, 0)),
        out_shape=jax.ShapeDtypeStruct((NP, HID), jnp.float32),
    )(acc, hs, cnt_t, b1)


def _tc3_body(a_ref, hs1_ref, cnt_ref, w2_ref, b2_ref, out_ref):
    c = cnt_ref[...]
    dinv = lax.rsqrt(1.0 + c[:, 0:1] + c[:, 1:2])
    a = a_ref[...]
    t = dinv * (a[0] + a[1] + hs1_ref[...])
    out_ref[...] = (jnp.dot(t, w2_ref[...], preferred_element_type=jnp.float32)
                    + b2_ref[...])


def _tc3(acc, hs1, cnt_t, W2, b2):
    return pl.pallas_call(
        _tc3_body,
        grid=(_GRID,),
        in_specs=[
            pl.BlockSpec((2, _R, HID), lambda r: (0, r, 0)),
            pl.BlockSpec((_R, HID), lambda r: (r, 0)),
            pl.BlockSpec((_R, 2), lambda r: (r, 0)),
            pl.BlockSpec((HID, NUM_CLASSES), lambda r: (0, 0)),
            pl.BlockSpec((NUM_CLASSES,), lambda r: (0,)),
        ],
        out_specs=pl.BlockSpec((_R, NUM_CLASSES), lambda r: (r, 0)),
        out_shape=jax.ShapeDtypeStruct((NP, NUM_CLASSES), jnp.float32),
    )(acc, hs1, cnt_t, W2, b2)


def kernel(x, edge_index, W1, b1, W2, b2):
    src = edge_index[0]
    dst = edge_index[1]
    pad = E_PAD - E
    src_p = jnp.concatenate(
        [src, jnp.zeros((pad,), jnp.int32)]).reshape(IDX_ROWS, 128)
    dst_p = jnp.concatenate(
        [dst, jnp.full((pad,), NP - 1, jnp.int32)]).reshape(IDX_ROWS, 128)
    x_p = jnp.pad(x, ((0, NP - N), (0, 0)))
    zeros_n = jnp.zeros((NP,), jnp.float32)
    zeros_nh = jnp.zeros((NP, HID), jnp.float32)
    ones128 = jnp.ones((128,), jnp.float32)

    cnt = _sc_degree(dst_p, zeros_n, ones128)
    cnt_t = jnp.transpose(cnt.reshape(2, NP), (1, 0))

    hs = _tc1(cnt_t, x_p, W1)
    acc1 = _sc_agg(hs, src_p, dst_p, zeros_nh).reshape(2, NP, HID)
    hs1 = _tc2(acc1, hs, cnt_t, b1)
    acc2 = _sc_agg(hs1, src_p, dst_p, zeros_nh).reshape(2, NP, HID)
    out = _tc3(acc2, hs1, cnt_t, W2, b2)
    return out[:N]
